# baseline (speedup 1.0000x reference)
import functools

import jax
import jax.numpy as jnp
from jax import lax
from jax.experimental import pallas as pl
from jax.experimental.pallas import tpu as pltpu
from jax.experimental.pallas import tpu_sc as plsc

N = 16 * 512 * 512
ROWS, COLS = 4096, 1024
BLK = 128
GRID = ROWS // BLK

NWORK = 16
NLANE = 16
SHARD = N // NWORK
SUB = SHARD // NLANE
CHUNK = 1024
NWIN = SUB // CHUNK
RADIX = 2048
NDIG = RADIX // NWORK
SHIFTS = (0, 11, 22)

_I32 = jnp.int32
_MIN32 = -2147483648


def _prep_body(pred_ref, true_ref, key_ref, g_ref):
    i = pl.program_id(0)
    s = pred_ref[...]
    g = true_ref[...]
    gf = g.astype(jnp.float32)
    e = 1.0 - s * (2.0 * gf - 1.0)
    bits = lax.bitcast_convert_type(e, _I32)
    m = jnp.where(bits < 0, ~bits, bits ^ _MIN32)
    key = (m & -2) | g
    key_ref[...] = ~key

    @pl.when(i == 0)
    def _():
        g_ref[0, 0] = 0.0

    g_ref[0, 0] += jnp.sum(gf)


def _prep(pred2d, true2d):
    return pl.pallas_call(
        _prep_body,
        grid=(GRID,),
        in_specs=[
            pl.BlockSpec((BLK, COLS), lambda i: (i, 0)),
            pl.BlockSpec((BLK, COLS), lambda i: (i, 0)),
        ],
        out_specs=[
            pl.BlockSpec((BLK, COLS), lambda i: (i, 0)),
            pl.BlockSpec(memory_space=pltpu.SMEM, block_shape=(1, 1), index_map=lambda i: (0, 0)),
        ],
        out_shape=[
            jax.ShapeDtypeStruct((ROWS, COLS), _I32),
            jax.ShapeDtypeStruct((1, 1), jnp.float32),
        ],
    )(pred2d, true2d)


def _sc_sort_body(in_hbm, out_hbm, tmp_hbm, win, tbl, scan_buf,
                  totals_v, totals_buf, base_v, grid_sh, totals_sh, sem_l, sem_s,
                  *bufs):
    wout_b = bufs[0:8]
    wdst_b = bufs[8:16]
    w = lax.axis_index("s")
    lanes = lax.iota(_I32, 16)
    ones16 = jnp.ones((16,), _I32)

    def load_window(src, t):
        hs = []
        for l in range(NLANE):
            base = w * SHARD + l * SUB + t * CHUNK
            hs.append(pltpu.async_copy(src.at[pl.ds(base, CHUNK)],
                                       win.at[pl.ds(l * CHUNK, CHUNK)], sem_l))
        for h in hs:
            h.wait()

    def digit_of(k, shift):
        return lax.shift_right_logical(k, shift) & (RADIX - 1)

    for shift, src, dst in ((SHIFTS[0], in_hbm, out_hbm),
                            (SHIFTS[1], out_hbm, tmp_hbm),
                            (SHIFTS[2], tmp_hbm, out_hbm)):
        def zero_body(i, _):
            tbl[pl.ds(pl.multiple_of(i * 16, 16), 16)] = jnp.zeros((16,), _I32)
            return 0

        lax.fori_loop(0, RADIX * 16 // 16, zero_body, 0)

        def hist_window(t, _):
            load_window(src, t)

            def hist_vec(r, _):
                for j in range(8):
                    v = r * 8 + j
                    k = plsc.load_gather(win, [lanes * CHUNK + v])
                    idx = digit_of(k, shift) * 16 + lanes
                    plsc.addupdate_scatter(tbl, [idx], ones16)
                return 0

            lax.fori_loop(0, CHUNK // 8, hist_vec, 0)
            return 0

        lax.fori_loop(0, NWIN, hist_window, 0)

        pltpu.sync_copy(tbl, grid_sh.at[pl.ds(w * (RADIX * 16), RADIX * 16)])
        plsc.subcore_barrier()

        for w2 in range(NWORK):
            pltpu.sync_copy(
                grid_sh.at[pl.ds(w2 * (RADIX * 16) + w * (NDIG * 16), NDIG * 16)],
                scan_buf.at[pl.ds(w2 * (NDIG * 16), NDIG * 16)])

        def scan_group(dg, _):
            def scan_pe(pe, carry):
                w2 = pe >> 4
                l = pe & 15
                idx = w2 * (NDIG * 16) + dg * 256 + lanes * 16 + l
                cnt = plsc.load_gather(scan_buf, [idx])
                plsc.store_scatter(scan_buf, [idx], carry)
                return carry + cnt

            carry = lax.fori_loop(0, NWORK * NLANE, scan_pe, jnp.zeros((16,), _I32))
            totals_v[pl.ds(pl.multiple_of(dg * 16, 16), 16)] = carry
            return 0

        lax.fori_loop(0, NDIG // 16, scan_group, 0)

        for w2 in range(NWORK):
            pltpu.sync_copy(
                scan_buf.at[pl.ds(w2 * (NDIG * 16), NDIG * 16)],
                grid_sh.at[pl.ds(w2 * (RADIX * 16) + w * (NDIG * 16), NDIG * 16)])
        pltpu.sync_copy(totals_v, totals_sh.at[pl.ds(w * NDIG, NDIG)])
        plsc.subcore_barrier()

        pltpu.sync_copy(totals_sh, totals_buf)

        def base_body(i, carry):
            v = totals_buf[pl.ds(pl.multiple_of(i * 16, 16), 16)]
            excl = plsc.cumsum(v) - v + carry
            base_v[pl.ds(pl.multiple_of(i * 16, 16), 16)] = excl
            return carry + jnp.sum(v)

        lax.fori_loop(0, RADIX // 16, base_body, jnp.zeros((), _I32))

        pltpu.sync_copy(grid_sh.at[pl.ds(w * (RADIX * 16), RADIX * 16)], tbl)

        def addbase_body(i, _):
            b16 = plsc.load_gather(base_v, [jnp.zeros((16,), _I32) + i])
            sl = pl.ds(pl.multiple_of(i * 16, 16), 16)
            tbl[sl] = tbl[sl] + b16
            return 0

        lax.fori_loop(0, RADIX, addbase_body, 0)
        plsc.subcore_barrier()

        def perm_window(t, _):
            load_window(src, t)

            def perm_group(rr, _):
                hs = []
                for b in range(8):
                    for j in range(8):
                        v = rr * 64 + b * 8 + j
                        k = plsc.load_gather(win, [lanes * CHUNK + v])
                        idx = digit_of(k, shift) * 16 + lanes
                        dv = plsc.load_gather(tbl, [idx])
                        plsc.addupdate_scatter(tbl, [idx], ones16)
                        wout_b[b][pl.ds(j * 16, 16)] = k
                        wdst_b[b][pl.ds(j * 16, 16)] = dv
                    hs.append(pltpu.async_copy(wout_b[b], dst.at[wdst_b[b]], sem_s))
                for h in hs:
                    h.wait()
                return 0

            lax.fori_loop(0, CHUNK // 64, perm_group, 0)
            return 0

        lax.fori_loop(0, NWIN, perm_window, 0)
        plsc.subcore_barrier()


def _sc_sort(key1d):
    mesh = plsc.VectorSubcoreMesh(core_axis_name="c", subcore_axis_name="s",
                                  num_cores=1, num_subcores=NWORK)
    fn = pl.kernel(
        _sc_sort_body,
        out_type=[jax.ShapeDtypeStruct((N,), _I32),
                  jax.ShapeDtypeStruct((N,), _I32)],
        mesh=mesh,
        compiler_params=pltpu.CompilerParams(needs_layout_passes=False),
        scratch_types=[
            pltpu.VMEM((NLANE * CHUNK,), _I32),
            pltpu.VMEM((RADIX * 16,), _I32),
            pltpu.VMEM((NWORK * NDIG * 16,), _I32),
            pltpu.VMEM((NDIG,), _I32),
            pltpu.VMEM((RADIX,), _I32),
            pltpu.VMEM((RADIX,), _I32),
            pltpu.VMEM_SHARED((NWORK * RADIX * 16,), _I32),
            pltpu.VMEM_SHARED((RADIX,), _I32),
            pltpu.SemaphoreType.DMA,
            pltpu.SemaphoreType.DMA,
        ] + [pltpu.VMEM((128,), _I32) for _ in range(16)],
    )
    out, _ = fn(key1d)
    return out


def _final_body(key_ref, g_ref, out_ref, cpref, acc):
    i = pl.program_id(0)

    @pl.when(i == 0)
    def _():
        cpref[0] = 0.0
        acc[0] = 0.0

    k = ~key_ref[...]
    g = (k & 1).astype(jnp.float32)
    m = k & -2
    bits = jnp.where(m < 0, m ^ _MIN32, ~m)
    e = lax.bitcast_convert_type(bits, jnp.float32)
    relu = jnp.maximum(e, 0.0)

    c1 = lax.broadcasted_iota(_I32, (COLS, COLS), 0)
    c2 = lax.broadcasted_iota(_I32, (COLS, COLS), 1)
    tri = (c1 <= c2).astype(jnp.float32)
    cs = jax.lax.dot(g, tri, precision=jax.lax.Precision.HIGHEST)
    rowsum = cs[:, COLS - 1:COLS]
    r1 = lax.broadcasted_iota(_I32, (BLK, BLK), 0)
    r2 = lax.broadcasted_iota(_I32, (BLK, BLK), 1)
    tri_s = (r2 < r1).astype(jnp.float32)
    rowpref = jax.lax.dot(tri_s, rowsum, precision=jax.lax.Precision.HIGHEST)

    G = g_ref[0, 0]
    C = cs + rowpref + cpref[0]
    ridx = lax.broadcasted_iota(_I32, (BLK, COLS), 0)
    cidx = lax.broadcasted_iota(_I32, (BLK, COLS), 1)
    idx = ((i * (BLK * COLS)) + ridx * COLS + cidx).astype(jnp.float32)

    jac_i = 1.0 - (G - C) / (G + (idx + 1.0) - C)
    Cp = C - g
    jac_p = jnp.where(idx == 0.0, 0.0, 1.0 - (G - Cp) / (G + idx - Cp))
    acc[0] += jnp.sum(relu * (jac_i - jac_p))
    cpref[0] += rowpref[BLK - 1, 0] + rowsum[BLK - 1, 0]

    @pl.when(i == GRID - 1)
    def _():
        out_ref[0, 0] = acc[0]


def _final(sorted2d, gtot):
    return pl.pallas_call(
        _final_body,
        grid=(GRID,),
        in_specs=[
            pl.BlockSpec((BLK, COLS), lambda i: (i, 0)),
            pl.BlockSpec(memory_space=pltpu.SMEM, block_shape=(1, 1), index_map=lambda i: (0, 0)),
        ],
        out_specs=pl.BlockSpec(memory_space=pltpu.SMEM, block_shape=(1, 1), index_map=lambda i: (0, 0)),
        out_shape=jax.ShapeDtypeStruct((1, 1), jnp.float32),
        scratch_shapes=[pltpu.SMEM((1,), jnp.float32), pltpu.SMEM((1,), jnp.float32)],
    )(sorted2d, gtot)


def kernel(y_pred, y_true):
    pred2d = y_pred.reshape(ROWS, COLS)
    true2d = y_true.reshape(ROWS, COLS).astype(_I32)
    key2d, gtot = _prep(pred2d, true2d)
    sorted1d = _sc_sort(key2d.reshape(N))
    loss = _final(sorted1d.reshape(ROWS, COLS), gtot)
    return loss[0, 0]


# speedup vs baseline: 1.4884x; 1.4884x over previous
import functools

import jax
import jax.numpy as jnp
from jax import lax
from jax.experimental import pallas as pl
from jax.experimental.pallas import tpu as pltpu
from jax.experimental.pallas import tpu_sc as plsc

N = 16 * 512 * 512
ROWS, COLS = 4096, 1024
BLK = 128
GRID = ROWS // BLK

NWORK = 16
NLANE = 16
SHARD = N // NWORK
SUB = SHARD // NLANE
CHUNK = 1024
NWIN = SUB // CHUNK
RADIX = 2048
NDIG = RADIX // NWORK
SHIFTS = (10, 21)

_I32 = jnp.int32
_MIN32 = -2147483648


def _prep_body(pred_ref, true_ref, key_ref, g_ref):
    i = pl.program_id(0)
    s = pred_ref[...]
    g = true_ref[...]
    gf = g.astype(jnp.float32)
    e = 1.0 - s * (2.0 * gf - 1.0)
    bits = lax.bitcast_convert_type(e, _I32)
    m = jnp.where(bits < 0, ~bits, bits ^ _MIN32)
    key = (m & -2) | g
    key_ref[...] = ~key

    @pl.when(i == 0)
    def _():
        g_ref[0, 0] = 0.0

    g_ref[0, 0] += jnp.sum(gf)


def _prep(pred2d, true2d):
    return pl.pallas_call(
        _prep_body,
        grid=(GRID,),
        in_specs=[
            pl.BlockSpec((BLK, COLS), lambda i: (i, 0)),
            pl.BlockSpec((BLK, COLS), lambda i: (i, 0)),
        ],
        out_specs=[
            pl.BlockSpec((BLK, COLS), lambda i: (i, 0)),
            pl.BlockSpec(memory_space=pltpu.SMEM, block_shape=(1, 1), index_map=lambda i: (0, 0)),
        ],
        out_shape=[
            jax.ShapeDtypeStruct((ROWS, COLS), _I32),
            jax.ShapeDtypeStruct((1, 1), jnp.float32),
        ],
    )(pred2d, true2d)


def _sc_sort_body(in_hbm, out_hbm, tmp_hbm, win, tbl, scan_buf,
                  totals_v, totals_buf, base_v, grid_sh, totals_sh, sem_l, sem_s,
                  *bufs):
    wout_A = bufs[0:8]
    wdst_A = bufs[8:16]
    wout_B = bufs[16:24]
    wdst_B = bufs[24:32]
    w = lax.axis_index("s")
    lanes = lax.iota(_I32, 16)
    ones16 = jnp.ones((16,), _I32)

    def load_window(src, t):
        hs = []
        for l in range(NLANE):
            base = w * SHARD + l * SUB + t * CHUNK
            hs.append(pltpu.async_copy(src.at[pl.ds(base, CHUNK)],
                                       win.at[pl.ds(l * CHUNK, CHUNK)], sem_l))
        for h in hs:
            h.wait()

    def digit_of(k, shift):
        return lax.shift_right_logical(k, shift) & (RADIX - 1)

    for shift, src, dst in ((SHIFTS[0], in_hbm, tmp_hbm),
                            (SHIFTS[1], tmp_hbm, out_hbm)):
        def zero_body(i, _):
            tbl[pl.ds(pl.multiple_of(i * 16, 16), 16)] = jnp.zeros((16,), _I32)
            return 0

        lax.fori_loop(0, RADIX * 16 // 16, zero_body, 0)

        def hist_window(t, _):
            load_window(src, t)

            def hist_vec(r, _):
                for j in range(8):
                    v = r * 8 + j
                    k = plsc.load_gather(win, [lanes * CHUNK + v])
                    idx = digit_of(k, shift) * 16 + lanes
                    plsc.addupdate_scatter(tbl, [idx], ones16)
                return 0

            lax.fori_loop(0, CHUNK // 8, hist_vec, 0)
            return 0

        lax.fori_loop(0, NWIN, hist_window, 0)

        pltpu.sync_copy(tbl, grid_sh.at[pl.ds(w * (RADIX * 16), RADIX * 16)])
        plsc.subcore_barrier()

        for w2 in range(NWORK):
            pltpu.sync_copy(
                grid_sh.at[pl.ds(w2 * (RADIX * 16) + w * (NDIG * 16), NDIG * 16)],
                scan_buf.at[pl.ds(w2 * (NDIG * 16), NDIG * 16)])

        def scan_group(dg, _):
            def scan_pe(pe, carry):
                w2 = pe >> 4
                l = pe & 15
                idx = w2 * (NDIG * 16) + dg * 256 + lanes * 16 + l
                cnt = plsc.load_gather(scan_buf, [idx])
                plsc.store_scatter(scan_buf, [idx], carry)
                return carry + cnt

            carry = lax.fori_loop(0, NWORK * NLANE, scan_pe, jnp.zeros((16,), _I32))
            totals_v[pl.ds(pl.multiple_of(dg * 16, 16), 16)] = carry
            return 0

        lax.fori_loop(0, NDIG // 16, scan_group, 0)

        for w2 in range(NWORK):
            pltpu.sync_copy(
                scan_buf.at[pl.ds(w2 * (NDIG * 16), NDIG * 16)],
                grid_sh.at[pl.ds(w2 * (RADIX * 16) + w * (NDIG * 16), NDIG * 16)])
        pltpu.sync_copy(totals_v, totals_sh.at[pl.ds(w * NDIG, NDIG)])
        plsc.subcore_barrier()

        pltpu.sync_copy(totals_sh, totals_buf)

        def base_body(i, carry):
            v = totals_buf[pl.ds(pl.multiple_of(i * 16, 16), 16)]
            excl = plsc.cumsum(v) - v + carry
            base_v[pl.ds(pl.multiple_of(i * 16, 16), 16)] = excl
            return carry + jnp.sum(v)

        lax.fori_loop(0, RADIX // 16, base_body, jnp.zeros((), _I32))

        pltpu.sync_copy(grid_sh.at[pl.ds(w * (RADIX * 16), RADIX * 16)], tbl)

        def addbase_body(i, _):
            b16 = plsc.load_gather(base_v, [jnp.zeros((16,), _I32) + i])
            sl = pl.ds(pl.multiple_of(i * 16, 16), 16)
            tbl[sl] = tbl[sl] + b16
            return 0

        lax.fori_loop(0, RADIX, addbase_body, 0)
        plsc.subcore_barrier()

        def drain_4k():
            pltpu.make_async_copy(src.at[pl.ds(0, 1024)],
                                  win.at[pl.ds(0, 1024)], sem_s).wait()

        def perm_half(base_v_idx, wout_s, wdst_s):
            for b in range(8):
                for j in range(8):
                    v = base_v_idx + b * 8 + j
                    k = plsc.load_gather(win, [lanes * CHUNK + v])
                    idx = digit_of(k, shift) * 16 + lanes
                    dv = plsc.load_gather(tbl, [idx])
                    plsc.addupdate_scatter(tbl, [idx], ones16)
                    wout_s[b][pl.ds(j * 16, 16)] = k
                    wdst_s[b][pl.ds(j * 16, 16)] = dv
                pltpu.async_copy(wout_s[b], dst.at[wdst_s[b]], sem_s)

        def perm_window(t, _):
            load_window(src, t)

            def perm_iter(rr, _):
                @pl.when(rr >= 1)
                def _():
                    drain_4k()

                perm_half(rr * 128, wout_A, wdst_A)

                @pl.when(rr >= 1)
                def _():
                    drain_4k()

                perm_half(rr * 128 + 64, wout_B, wdst_B)
                return 0

            lax.fori_loop(0, CHUNK // 128, perm_iter, 0)
            drain_4k()
            drain_4k()
            return 0

        lax.fori_loop(0, NWIN, perm_window, 0)
        plsc.subcore_barrier()


def _sc_sort(key1d):
    mesh = plsc.VectorSubcoreMesh(core_axis_name="c", subcore_axis_name="s",
                                  num_cores=1, num_subcores=NWORK)
    fn = pl.kernel(
        _sc_sort_body,
        out_type=[jax.ShapeDtypeStruct((N,), _I32),
                  jax.ShapeDtypeStruct((N,), _I32)],
        mesh=mesh,
        compiler_params=pltpu.CompilerParams(needs_layout_passes=False),
        scratch_types=[
            pltpu.VMEM((NLANE * CHUNK,), _I32),
            pltpu.VMEM((RADIX * 16,), _I32),
            pltpu.VMEM((NWORK * NDIG * 16,), _I32),
            pltpu.VMEM((NDIG,), _I32),
            pltpu.VMEM((RADIX,), _I32),
            pltpu.VMEM((RADIX,), _I32),
            pltpu.VMEM_SHARED((NWORK * RADIX * 16,), _I32),
            pltpu.VMEM_SHARED((RADIX,), _I32),
            pltpu.SemaphoreType.DMA,
            pltpu.SemaphoreType.DMA,
        ] + [pltpu.VMEM((128,), _I32) for _ in range(32)],
    )
    out, _ = fn(key1d)
    return out


def _final_body(key_ref, g_ref, out_ref, cpref, acc):
    i = pl.program_id(0)

    @pl.when(i == 0)
    def _():
        cpref[0] = 0.0
        acc[0] = 0.0

    k = ~key_ref[...]
    g = (k & 1).astype(jnp.float32)
    m = k & -2
    bits = jnp.where(m < 0, m ^ _MIN32, ~m)
    e = lax.bitcast_convert_type(bits, jnp.float32)
    relu = jnp.maximum(e, 0.0)

    c1 = lax.broadcasted_iota(_I32, (COLS, COLS), 0)
    c2 = lax.broadcasted_iota(_I32, (COLS, COLS), 1)
    tri = (c1 <= c2).astype(jnp.float32)
    cs = jax.lax.dot(g, tri, precision=jax.lax.Precision.HIGHEST)
    rowsum = cs[:, COLS - 1:COLS]
    r1 = lax.broadcasted_iota(_I32, (BLK, BLK), 0)
    r2 = lax.broadcasted_iota(_I32, (BLK, BLK), 1)
    tri_s = (r2 < r1).astype(jnp.float32)
    rowpref = jax.lax.dot(tri_s, rowsum, precision=jax.lax.Precision.HIGHEST)

    G = g_ref[0, 0]
    C = cs + rowpref + cpref[0]
    ridx = lax.broadcasted_iota(_I32, (BLK, COLS), 0)
    cidx = lax.broadcasted_iota(_I32, (BLK, COLS), 1)
    idx = ((i * (BLK * COLS)) + ridx * COLS + cidx).astype(jnp.float32)

    jac_i = 1.0 - (G - C) / (G + (idx + 1.0) - C)
    Cp = C - g
    jac_p = jnp.where(idx == 0.0, 0.0, 1.0 - (G - Cp) / (G + idx - Cp))
    acc[0] += jnp.sum(relu * (jac_i - jac_p))
    cpref[0] += rowpref[BLK - 1, 0] + rowsum[BLK - 1, 0]

    @pl.when(i == GRID - 1)
    def _():
        out_ref[0, 0] = acc[0]


def _final(sorted2d, gtot):
    return pl.pallas_call(
        _final_body,
        grid=(GRID,),
        in_specs=[
            pl.BlockSpec((BLK, COLS), lambda i: (i, 0)),
            pl.BlockSpec(memory_space=pltpu.SMEM, block_shape=(1, 1), index_map=lambda i: (0, 0)),
        ],
        out_specs=pl.BlockSpec(memory_space=pltpu.SMEM, block_shape=(1, 1), index_map=lambda i: (0, 0)),
        out_shape=jax.ShapeDtypeStruct((1, 1), jnp.float32),
        scratch_shapes=[pltpu.SMEM((1,), jnp.float32), pltpu.SMEM((1,), jnp.float32)],
    )(sorted2d, gtot)


def kernel(y_pred, y_true):
    pred2d = y_pred.reshape(ROWS, COLS)
    true2d = y_true.reshape(ROWS, COLS).astype(_I32)
    key2d, gtot = _prep(pred2d, true2d)
    sorted1d = _sc_sort(key2d.reshape(N))
    loss = _final(sorted1d.reshape(ROWS, COLS), gtot)
    return loss[0, 0]


# speedup vs baseline: 1.5528x; 1.0432x over previous
import functools

import jax
import jax.numpy as jnp
from jax import lax
from jax.experimental import pallas as pl
from jax.experimental.pallas import tpu as pltpu
from jax.experimental.pallas import tpu_sc as plsc

N = 16 * 512 * 512
ROWS, COLS = 4096, 1024
BLK = 128
GRID = ROWS // BLK

NWORK = 16
NLANE = 16
SHARD = N // NWORK
SUB = SHARD // NLANE
CHUNK = 1024
NWIN = SUB // CHUNK
RADIX = 2048
NDIG = RADIX // NWORK
SHIFTS = (10, 21)

_I32 = jnp.int32
_MIN32 = -2147483648


def _prep_body(pred_ref, true_ref, key_ref, g_ref):
    i = pl.program_id(0)
    s = pred_ref[...]
    g = true_ref[...]
    gf = g.astype(jnp.float32)
    e = 1.0 - s * (2.0 * gf - 1.0)
    bits = lax.bitcast_convert_type(e, _I32)
    m = jnp.where(bits < 0, ~bits, bits ^ _MIN32)
    key = (m & -2) | g
    key_ref[...] = ~key

    @pl.when(i == 0)
    def _():
        g_ref[0, 0] = 0.0

    g_ref[0, 0] += jnp.sum(gf)


def _prep(pred2d, true2d):
    return pl.pallas_call(
        _prep_body,
        grid=(GRID,),
        in_specs=[
            pl.BlockSpec((BLK, COLS), lambda i: (i, 0)),
            pl.BlockSpec((BLK, COLS), lambda i: (i, 0)),
        ],
        out_specs=[
            pl.BlockSpec((BLK, COLS), lambda i: (i, 0)),
            pl.BlockSpec(memory_space=pltpu.SMEM, block_shape=(1, 1), index_map=lambda i: (0, 0)),
        ],
        out_shape=[
            jax.ShapeDtypeStruct((ROWS, COLS), _I32),
            jax.ShapeDtypeStruct((1, 1), jnp.float32),
        ],
    )(pred2d, true2d)


NC = 2
NW32 = NC * NWORK
SHARD32 = N // NW32
SUB32 = SHARD32 // NLANE
NWIN32 = SUB32 // CHUNK
GRIDW = RADIX * 16
NDIG32 = RADIX // NW32

_mesh = None


def _get_mesh():
    global _mesh
    if _mesh is None:
        _mesh = plsc.VectorSubcoreMesh(core_axis_name="c", subcore_axis_name="s",
                                       num_cores=NC, num_subcores=NWORK)
    return _mesh


def _wid():
    return lax.axis_index("s") * NC + lax.axis_index("c")


def _digit_of(k, shift):
    return lax.shift_right_logical(k, shift) & (RADIX - 1)


def _load_window(src, win, w, t, sem):
    hs = []
    for l in range(NLANE):
        base = w * SHARD32 + l * SUB32 + t * CHUNK
        hs.append(pltpu.async_copy(src.at[pl.ds(base, CHUNK)],
                                   win.at[pl.ds(l * CHUNK, CHUNK)], sem))
    for h in hs:
        h.wait()


def _hist_body(shift):
    def body(src, grid, win, tbl, sem):
        w = _wid()
        lanes = lax.iota(_I32, 16)
        ones16 = jnp.ones((16,), _I32)

        def zero_body(i, _):
            tbl[pl.ds(pl.multiple_of(i * 16, 16), 16)] = jnp.zeros((16,), _I32)
            return 0

        lax.fori_loop(0, GRIDW // 16, zero_body, 0)

        def hist_window(t, _):
            _load_window(src, win, w, t, sem)

            def hist_vec(r, _):
                for j in range(8):
                    v = r * 8 + j
                    k = plsc.load_gather(win, [lanes * CHUNK + v])
                    idx = _digit_of(k, shift) * 16 + lanes
                    plsc.addupdate_scatter(tbl, [idx], ones16)
                return 0

            lax.fori_loop(0, CHUNK // 8, hist_vec, 0)
            return 0

        lax.fori_loop(0, NWIN32, hist_window, 0)
        pltpu.sync_copy(tbl, grid.at[pl.ds(w * GRIDW, GRIDW)])

    return body


def _scan_body(src_unused, grid, offs, totals, scan_buf, totals_v, sem):
    w = _wid()
    lanes = lax.iota(_I32, 16)
    for w2 in range(NW32):
        pltpu.sync_copy(grid.at[pl.ds(w2 * GRIDW + w * (NDIG32 * 16), NDIG32 * 16)],
                        scan_buf.at[pl.ds(w2 * (NDIG32 * 16), NDIG32 * 16)])

    def scan_group(dg, _):
        def scan_pe(pe, carry):
            w2 = pe >> 4
            l = pe & 15
            idx = w2 * (NDIG32 * 16) + dg * 256 + lanes * 16 + l
            cnt = plsc.load_gather(scan_buf, [idx])
            plsc.store_scatter(scan_buf, [idx], carry)
            return carry + cnt

        carry = lax.fori_loop(0, NW32 * NLANE, scan_pe, jnp.zeros((16,), _I32))
        totals_v[pl.ds(pl.multiple_of(dg * 16, 16), 16)] = carry
        return 0

    lax.fori_loop(0, NDIG32 // 16, scan_group, 0)

    for w2 in range(NW32):
        pltpu.sync_copy(scan_buf.at[pl.ds(w2 * (NDIG32 * 16), NDIG32 * 16)],
                        offs.at[pl.ds(w2 * GRIDW + w * (NDIG32 * 16), NDIG32 * 16)])
    pltpu.sync_copy(totals_v, totals.at[pl.ds(w * NDIG32, NDIG32)])


def _perm_body(shift):
    def body(src, offs, totals, dst, win, tbl, totals_buf, base_v, sem_l, sem_s,
             *bufs):
        wout_A = bufs[0:8]
        wdst_A = bufs[8:16]
        wout_B = bufs[16:24]
        wdst_B = bufs[24:32]
        w = _wid()
        lanes = lax.iota(_I32, 16)
        ones16 = jnp.ones((16,), _I32)

        pltpu.sync_copy(offs.at[pl.ds(w * GRIDW, GRIDW)], tbl)
        pltpu.sync_copy(totals, totals_buf)

        def base_body(i, carry):
            v = totals_buf[pl.ds(pl.multiple_of(i * 16, 16), 16)]
            excl = plsc.cumsum(v) - v + carry
            base_v[pl.ds(pl.multiple_of(i * 16, 16), 16)] = excl
            return carry + jnp.sum(v)

        lax.fori_loop(0, RADIX // 16, base_body, jnp.zeros((), _I32))

        def addbase_body(i, _):
            b16 = plsc.load_gather(base_v, [jnp.zeros((16,), _I32) + i])
            sl = pl.ds(pl.multiple_of(i * 16, 16), 16)
            tbl[sl] = tbl[sl] + b16
            return 0

        lax.fori_loop(0, RADIX, addbase_body, 0)

        def drain_4k():
            pltpu.make_async_copy(src.at[pl.ds(0, 1024)],
                                  win.at[pl.ds(0, 1024)], sem_s).wait()

        def perm_half(base_v_idx, wout_s, wdst_s):
            for b in range(8):
                for j in range(8):
                    v = base_v_idx + b * 8 + j
                    k = plsc.load_gather(win, [lanes * CHUNK + v])
                    idx = _digit_of(k, shift) * 16 + lanes
                    dv = plsc.load_gather(tbl, [idx])
                    plsc.addupdate_scatter(tbl, [idx], ones16)
                    wout_s[b][pl.ds(j * 16, 16)] = k
                    wdst_s[b][pl.ds(j * 16, 16)] = dv
                pltpu.async_copy(wout_s[b], dst.at[wdst_s[b]], sem_s)

        def perm_window(t, _):
            _load_window(src, win, w, t, sem_l)

            def perm_iter(rr, _):
                @pl.when(rr >= 1)
                def _():
                    drain_4k()

                perm_half(rr * 128, wout_A, wdst_A)

                @pl.when(rr >= 1)
                def _():
                    drain_4k()

                perm_half(rr * 128 + 64, wout_B, wdst_B)
                return 0

            lax.fori_loop(0, CHUNK // 128, perm_iter, 0)
            drain_4k()
            drain_4k()
            return 0

        lax.fori_loop(0, NWIN32, perm_window, 0)

    return body


def _sc_sort(key1d):
    mesh = _get_mesh()
    cp = pltpu.CompilerParams(needs_layout_passes=False)
    i32 = _I32

    def hist_call(src, shift):
        fn = pl.kernel(
            _hist_body(shift),
            out_type=jax.ShapeDtypeStruct((NW32 * GRIDW,), i32),
            mesh=mesh, compiler_params=cp,
            scratch_types=[
                pltpu.VMEM((NLANE * CHUNK,), i32),
                pltpu.VMEM((GRIDW,), i32),
                pltpu.SemaphoreType.DMA,
            ],
        )
        return fn(src)

    def scan_call(src, grid):
        fn = pl.kernel(
            _scan_body,
            out_type=[jax.ShapeDtypeStruct((NW32 * GRIDW,), i32),
                      jax.ShapeDtypeStruct((RADIX,), i32)],
            mesh=mesh, compiler_params=cp,
            scratch_types=[
                pltpu.VMEM((NW32 * NDIG32 * 16,), i32),
                pltpu.VMEM((NDIG32,), i32),
                pltpu.SemaphoreType.DMA,
            ],
        )
        return fn(src, grid)

    def perm_call(src, offs, totals, shift):
        fn = pl.kernel(
            _perm_body(shift),
            out_type=jax.ShapeDtypeStruct((N,), i32),
            mesh=mesh, compiler_params=cp,
            scratch_types=[
                pltpu.VMEM((NLANE * CHUNK,), i32),
                pltpu.VMEM((GRIDW,), i32),
                pltpu.VMEM((RADIX,), i32),
                pltpu.VMEM((RADIX,), i32),
                pltpu.SemaphoreType.DMA,
                pltpu.SemaphoreType.DMA,
            ] + [pltpu.VMEM((128,), i32) for _ in range(32)],
        )
        return fn(src, offs, totals)

    cur = key1d
    for shift in SHIFTS:
        grid = hist_call(cur, shift)
        offs, totals = scan_call(cur, grid)
        cur = perm_call(cur, offs, totals, shift)
    return cur


def _final_body(key_ref, g_ref, out_ref, cpref, acc):
    i = pl.program_id(0)

    @pl.when(i == 0)
    def _():
        cpref[0] = 0.0
        acc[0] = 0.0

    k = ~key_ref[...]
    g = (k & 1).astype(jnp.float32)
    m = k & -2
    bits = jnp.where(m < 0, m ^ _MIN32, ~m)
    e = lax.bitcast_convert_type(bits, jnp.float32)
    relu = jnp.maximum(e, 0.0)

    c1 = lax.broadcasted_iota(_I32, (COLS, COLS), 0)
    c2 = lax.broadcasted_iota(_I32, (COLS, COLS), 1)
    tri = (c1 <= c2).astype(jnp.float32)
    cs = jax.lax.dot(g, tri, precision=jax.lax.Precision.HIGHEST)
    rowsum = cs[:, COLS - 1:COLS]
    r1 = lax.broadcasted_iota(_I32, (BLK, BLK), 0)
    r2 = lax.broadcasted_iota(_I32, (BLK, BLK), 1)
    tri_s = (r2 < r1).astype(jnp.float32)
    rowpref = jax.lax.dot(tri_s, rowsum, precision=jax.lax.Precision.HIGHEST)

    G = g_ref[0, 0]
    C = cs + rowpref + cpref[0]
    ridx = lax.broadcasted_iota(_I32, (BLK, COLS), 0)
    cidx = lax.broadcasted_iota(_I32, (BLK, COLS), 1)
    idx = ((i * (BLK * COLS)) + ridx * COLS + cidx).astype(jnp.float32)

    jac_i = 1.0 - (G - C) / (G + (idx + 1.0) - C)
    Cp = C - g
    jac_p = jnp.where(idx == 0.0, 0.0, 1.0 - (G - Cp) / (G + idx - Cp))
    acc[0] += jnp.sum(relu * (jac_i - jac_p))
    cpref[0] += rowpref[BLK - 1, 0] + rowsum[BLK - 1, 0]

    @pl.when(i == GRID - 1)
    def _():
        out_ref[0, 0] = acc[0]


def _final(sorted2d, gtot):
    return pl.pallas_call(
        _final_body,
        grid=(GRID,),
        in_specs=[
            pl.BlockSpec((BLK, COLS), lambda i: (i, 0)),
            pl.BlockSpec(memory_space=pltpu.SMEM, block_shape=(1, 1), index_map=lambda i: (0, 0)),
        ],
        out_specs=pl.BlockSpec(memory_space=pltpu.SMEM, block_shape=(1, 1), index_map=lambda i: (0, 0)),
        out_shape=jax.ShapeDtypeStruct((1, 1), jnp.float32),
        scratch_shapes=[pltpu.SMEM((1,), jnp.float32), pltpu.SMEM((1,), jnp.float32)],
    )(sorted2d, gtot)


def kernel(y_pred, y_true):
    pred2d = y_pred.reshape(ROWS, COLS)
    true2d = y_true.reshape(ROWS, COLS).astype(_I32)
    key2d, gtot = _prep(pred2d, true2d)
    sorted1d = _sc_sort(key2d.reshape(N))
    loss = _final(sorted1d.reshape(ROWS, COLS), gtot)
    return loss[0, 0]


# speedup vs baseline: 3.6190x; 2.3306x over previous
import functools

import jax
import jax.numpy as jnp
from jax import lax
from jax.experimental import pallas as pl
from jax.experimental.pallas import tpu as pltpu
from jax.experimental.pallas import tpu_sc as plsc

N = 16 * 512 * 512
ROWS, COLS = 4096, 1024
BLK = 128
GRID = ROWS // BLK

NWORK = 16
NLANE = 16
SHARD = N // NWORK
SUB = SHARD // NLANE
CHUNK = 1024
NWIN = SUB // CHUNK
RADIX = 2048
NDIG = RADIX // NWORK
SHIFTS = (10, 21)

_I32 = jnp.int32
_MIN32 = -2147483648


def _prep_body(pred_ref, true_ref, key_ref, g_ref):
    i = pl.program_id(0)
    s = pred_ref[...]
    g = true_ref[...]
    gf = g.astype(jnp.float32)
    e = 1.0 - s * (2.0 * gf - 1.0)
    bits = lax.bitcast_convert_type(e, _I32)
    m = jnp.where(bits < 0, ~bits, bits ^ _MIN32)
    key = (m & -2) | g
    key_ref[...] = ~key

    @pl.when(i == 0)
    def _():
        g_ref[0, 0] = 0.0

    g_ref[0, 0] += jnp.sum(gf)


def _prep(pred2d, true2d):
    return pl.pallas_call(
        _prep_body,
        grid=(GRID,),
        in_specs=[
            pl.BlockSpec((BLK, COLS), lambda i: (i, 0)),
            pl.BlockSpec((BLK, COLS), lambda i: (i, 0)),
        ],
        out_specs=[
            pl.BlockSpec((BLK, COLS), lambda i: (i, 0)),
            pl.BlockSpec(memory_space=pltpu.SMEM, block_shape=(1, 1), index_map=lambda i: (0, 0)),
        ],
        out_shape=[
            jax.ShapeDtypeStruct((ROWS, COLS), _I32),
            jax.ShapeDtypeStruct((1, 1), jnp.float32),
        ],
    )(pred2d, true2d)


NC = 2
NW32 = NC * NWORK
SHARD32 = N // NW32
SUB32 = SHARD32 // NLANE
NWIN32 = SUB32 // CHUNK
GRIDW = RADIX * 16
NDIG32 = RADIX // NW32

_mesh = None


def _get_mesh():
    global _mesh
    if _mesh is None:
        _mesh = plsc.VectorSubcoreMesh(core_axis_name="c", subcore_axis_name="s",
                                       num_cores=NC, num_subcores=NWORK)
    return _mesh


def _wid():
    return lax.axis_index("s") * NC + lax.axis_index("c")


def _digit_of(k, shift):
    return lax.shift_right_logical(k, shift) & (RADIX - 1)


def _load_window(src, win, w, t, sem):
    hs = []
    for l in range(NLANE):
        base = w * SHARD32 + l * SUB32 + t * CHUNK
        hs.append(pltpu.async_copy(src.at[pl.ds(base, CHUNK)],
                                   win.at[pl.ds(l * CHUNK, CHUNK)], sem))
    for h in hs:
        h.wait()


def _hist_body(shift):
    def body(src, grid, win, tbl, sem):
        w = _wid()
        lanes = lax.iota(_I32, 16)
        ones16 = jnp.ones((16,), _I32)

        def zero_body(i, _):
            tbl[pl.ds(pl.multiple_of(i * 16, 16), 16)] = jnp.zeros((16,), _I32)
            return 0

        lax.fori_loop(0, GRIDW // 16, zero_body, 0)

        def hist_window(t, _):
            _load_window(src, win, w, t, sem)

            def hist_vec(r, _):
                for j in range(8):
                    v = r * 8 + j
                    k = plsc.load_gather(win, [lanes * CHUNK + v])
                    idx = _digit_of(k, shift) * 16 + lanes
                    plsc.addupdate_scatter(tbl, [idx], ones16)
                return 0

            lax.fori_loop(0, CHUNK // 8, hist_vec, 0)
            return 0

        lax.fori_loop(0, NWIN32, hist_window, 0)
        pltpu.sync_copy(tbl, grid.at[pl.ds(w * GRIDW, GRIDW)])

    return body


def _scan_body(src_unused, grid, offs, totals, scan_buf, totals_v, sem):
    w = _wid()
    lanes = lax.iota(_I32, 16)
    for w2 in range(NW32):
        pltpu.sync_copy(grid.at[pl.ds(w2 * GRIDW + w * (NDIG32 * 16), NDIG32 * 16)],
                        scan_buf.at[pl.ds(w2 * (NDIG32 * 16), NDIG32 * 16)])

    def scan_group(dg, _):
        def scan_pe(pe, carry):
            w2 = pe >> 4
            l = pe & 15
            idx = w2 * (NDIG32 * 16) + dg * 256 + lanes * 16 + l
            cnt = plsc.load_gather(scan_buf, [idx])
            plsc.store_scatter(scan_buf, [idx], carry)
            return carry + cnt

        carry = lax.fori_loop(0, NW32 * NLANE, scan_pe, jnp.zeros((16,), _I32))
        totals_v[pl.ds(pl.multiple_of(dg * 16, 16), 16)] = carry
        return 0

    lax.fori_loop(0, NDIG32 // 16, scan_group, 0)

    for w2 in range(NW32):
        pltpu.sync_copy(scan_buf.at[pl.ds(w2 * (NDIG32 * 16), NDIG32 * 16)],
                        offs.at[pl.ds(w2 * GRIDW + w * (NDIG32 * 16), NDIG32 * 16)])
    pltpu.sync_copy(totals_v, totals.at[pl.ds(w * NDIG32, NDIG32)])


H_ROUND = 630784
STAGE_SZ = 638976
NROUNDS = 7
NROUND_SC = 4
NTRASH = 256
FCHUNK = 8192
NFCH = H_ROUND // FCHUNK


def _perm_body(shift):
    def body(src, offs, totals, dst_hbm, win, tbl2, totals_buf, base_v, stage,
             sem_l, sem_s, *bufs):
        wout_A = bufs[0:8]
        wdst_A = bufs[8:16]
        wout_B = bufs[16:24]
        wdst_B = bufs[24:32]
        c = lax.axis_index("c")
        s16 = lax.axis_index("s")
        lanes = lax.iota(_I32, 16)
        ones16 = jnp.ones((16,), _I32)

        def load_counters():
            for sh in range(2):
                pltpu.sync_copy(offs.at[pl.ds((s16 * 2 + sh) * GRIDW, GRIDW)],
                                tbl2.at[pl.ds(sh * GRIDW, GRIDW)])
            pltpu.sync_copy(totals.at[pl.ds(0, RADIX)], totals_buf)

            def base_body(i, carry):
                v = totals_buf[pl.ds(pl.multiple_of(i * 16, 16), 16)]
                excl = plsc.cumsum(v) - v + carry
                base_v[pl.ds(pl.multiple_of(i * 16, 16), 16)] = excl
                return carry + jnp.sum(v)

            lax.fori_loop(0, RADIX // 16, base_body, jnp.zeros((), _I32))

            def addbase_body(i, _):
                b16 = plsc.load_gather(base_v, [jnp.zeros((16,), _I32) + (i & (RADIX - 1))])
                sl = pl.ds(pl.multiple_of(i * 16, 16), 16)
                tbl2[sl] = tbl2[sl] + b16
                return 0

            lax.fori_loop(0, 2 * RADIX, addbase_body, 0)

        def drain_4k():
            pltpu.make_async_copy(src.at[pl.ds(0, 1024)],
                                  win.at[pl.ds(0, 1024)], sem_s).wait()

        def sweep(i, _):
            rr = i // (2 * NWIN32)
            sh = (i // NWIN32) & 1
            t = i & (NWIN32 - 1)
            rnd = c * NROUND_SC + rr
            lo = rnd * H_ROUND
            live = rnd < NROUNDS

            @pl.when(live & ((i & (2 * NWIN32 - 1)) == 0))
            def _():
                load_counters()

            @pl.when(live)
            def _():
                shard = s16 * 2 + sh
                hs = []
                for l in range(NLANE):
                    base = shard * SHARD32 + l * SUB32 + t * CHUNK
                    hs.append(pltpu.async_copy(src.at[pl.ds(base, CHUNK)],
                                               win.at[pl.ds(l * CHUNK, CHUNK)], sem_l))
                for h in hs:
                    h.wait()

            def perm_half(base_v_idx, wout_s, wdst_s):
                for b in range(8):
                    for j in range(8):
                        v = base_v_idx + b * 8 + j
                        k = plsc.load_gather(win, [lanes * CHUNK + v])
                        idx = _digit_of(k, shift) * 16 + lanes + sh * GRIDW
                        dv = plsc.load_gather(tbl2, [idx])
                        plsc.addupdate_scatter(tbl2, [idx], ones16)
                        rel = dv - lo
                        inr = (rel >= 0) & (rel < H_ROUND)
                        spi = jnp.where(inr, rel, H_ROUND + (lanes * 16 + v) % NTRASH)
                        wout_s[b][pl.ds(j * 16, 16)] = k
                        wdst_s[b][pl.ds(j * 16, 16)] = spi
                    pltpu.async_copy(wout_s[b], stage.at[wdst_s[b]], sem_s)

            def perm_iter(rg, _):
                @pl.when(rg >= 1)
                def _():
                    drain_4k()

                perm_half(rg * 128, wout_A, wdst_A)

                @pl.when(rg >= 1)
                def _():
                    drain_4k()

                perm_half(rg * 128 + 64, wout_B, wdst_B)
                return 0

            @pl.when(live)
            def _():
                lax.fori_loop(0, CHUNK // 128, perm_iter, 0)
                drain_4k()
                drain_4k()

            @pl.when(live & ((i & (2 * NWIN32 - 1)) == (2 * NWIN32 - 1)))
            def _():
                plsc.subcore_barrier()
                def flush_q(q, _):
                    ch = s16 + q * NWORK
                    fbase = rnd * H_ROUND + ch * FCHUNK

                    @pl.when((ch < NFCH) & (fbase + FCHUNK <= N))
                    def _():
                        pltpu.sync_copy(
                            stage.at[pl.ds(ch * FCHUNK, FCHUNK)],
                            dst_hbm.at[pl.ds(fbase, FCHUNK)])
                    return 0

                lax.fori_loop(0, (NFCH + NWORK - 1) // NWORK, flush_q, 0)
                plsc.subcore_barrier()
            return 0

        lax.fori_loop(0, NROUND_SC * 2 * NWIN32, sweep, 0)

    return body


def _sc_sort(key1d):
    mesh = _get_mesh()
    cp = pltpu.CompilerParams(needs_layout_passes=False)
    i32 = _I32

    def hist_call(src, shift):
        fn = pl.kernel(
            _hist_body(shift),
            out_type=jax.ShapeDtypeStruct((NW32 * GRIDW,), i32),
            mesh=mesh, compiler_params=cp,
            scratch_types=[
                pltpu.VMEM((NLANE * CHUNK,), i32),
                pltpu.VMEM((GRIDW,), i32),
                pltpu.SemaphoreType.DMA,
            ],
        )
        return fn(src)

    def scan_call(src, grid):
        fn = pl.kernel(
            _scan_body,
            out_type=[jax.ShapeDtypeStruct((NW32 * GRIDW,), i32),
                      jax.ShapeDtypeStruct((RADIX,), i32)],
            mesh=mesh, compiler_params=cp,
            scratch_types=[
                pltpu.VMEM((NW32 * NDIG32 * 16,), i32),
                pltpu.VMEM((NDIG32,), i32),
                pltpu.SemaphoreType.DMA,
            ],
        )
        return fn(src, grid)

    def perm_call(src, offs, totals, shift):
        fn = pl.kernel(
            _perm_body(shift),
            out_type=jax.ShapeDtypeStruct((N,), i32),
            mesh=mesh, compiler_params=cp,
            scratch_types=[
                pltpu.VMEM((NLANE * CHUNK,), i32),
                pltpu.VMEM((2 * GRIDW,), i32),
                pltpu.VMEM((RADIX,), i32),
                pltpu.VMEM((RADIX,), i32),
                pltpu.VMEM_SHARED((STAGE_SZ,), i32),
                pltpu.SemaphoreType.DMA,
                pltpu.SemaphoreType.DMA,
            ] + [pltpu.VMEM((128,), i32) for _ in range(32)],
        )
        return fn(src, offs, totals)

    cur = key1d
    for shift in SHIFTS:
        grid = hist_call(cur, shift)
        offs, totals = scan_call(cur, grid)
        cur = perm_call(cur, offs, totals, shift)
    return cur


def _final_body(key_ref, g_ref, out_ref, cpref, acc):
    i = pl.program_id(0)

    @pl.when(i == 0)
    def _():
        cpref[0] = 0.0
        acc[0] = 0.0

    k = ~key_ref[...]
    g = (k & 1).astype(jnp.float32)
    m = k & -2
    bits = jnp.where(m < 0, m ^ _MIN32, ~m)
    e = lax.bitcast_convert_type(bits, jnp.float32)
    relu = jnp.maximum(e, 0.0)

    c1 = lax.broadcasted_iota(_I32, (COLS, COLS), 0)
    c2 = lax.broadcasted_iota(_I32, (COLS, COLS), 1)
    tri = (c1 <= c2).astype(jnp.float32)
    cs = jax.lax.dot(g, tri, precision=jax.lax.Precision.HIGHEST)
    rowsum = cs[:, COLS - 1:COLS]
    r1 = lax.broadcasted_iota(_I32, (BLK, BLK), 0)
    r2 = lax.broadcasted_iota(_I32, (BLK, BLK), 1)
    tri_s = (r2 < r1).astype(jnp.float32)
    rowpref = jax.lax.dot(tri_s, rowsum, precision=jax.lax.Precision.HIGHEST)

    G = g_ref[0, 0]
    C = cs + rowpref + cpref[0]
    ridx = lax.broadcasted_iota(_I32, (BLK, COLS), 0)
    cidx = lax.broadcasted_iota(_I32, (BLK, COLS), 1)
    idx = ((i * (BLK * COLS)) + ridx * COLS + cidx).astype(jnp.float32)

    jac_i = 1.0 - (G - C) / (G + (idx + 1.0) - C)
    Cp = C - g
    jac_p = jnp.where(idx == 0.0, 0.0, 1.0 - (G - Cp) / (G + idx - Cp))
    acc[0] += jnp.sum(relu * (jac_i - jac_p))
    cpref[0] += rowpref[BLK - 1, 0] + rowsum[BLK - 1, 0]

    @pl.when(i == GRID - 1)
    def _():
        out_ref[0, 0] = acc[0]


def _final(sorted2d, gtot):
    return pl.pallas_call(
        _final_body,
        grid=(GRID,),
        in_specs=[
            pl.BlockSpec((BLK, COLS), lambda i: (i, 0)),
            pl.BlockSpec(memory_space=pltpu.SMEM, block_shape=(1, 1), index_map=lambda i: (0, 0)),
        ],
        out_specs=pl.BlockSpec(memory_space=pltpu.SMEM, block_shape=(1, 1), index_map=lambda i: (0, 0)),
        out_shape=jax.ShapeDtypeStruct((1, 1), jnp.float32),
        scratch_shapes=[pltpu.SMEM((1,), jnp.float32), pltpu.SMEM((1,), jnp.float32)],
    )(sorted2d, gtot)


def kernel(y_pred, y_true):
    pred2d = y_pred.reshape(ROWS, COLS)
    true2d = y_true.reshape(ROWS, COLS).astype(_I32)
    key2d, gtot = _prep(pred2d, true2d)
    sorted1d = _sc_sort(key2d.reshape(N))
    loss = _final(sorted1d.reshape(ROWS, COLS), gtot)
    return loss[0, 0]
